# Initial kernel scaffold; baseline (speedup 1.0000x reference)
import functools

import jax
import jax.numpy as jnp
from jax import lax
from jax.experimental import pallas as pl
from jax.experimental.pallas import tpu as pltpu
from jax.experimental.pallas import tpu_sc as plsc

B = 16384
MUT_SLOTS = 20
CH = 128
NW = 32
CPW = (B // NW) // CH
RPW = B // NW

MAP_DIM = 32
CMD_DIM = 48
MUT_DIM = 48
AI_DIM = 16


def _sc_gather(map_arr, cmd_arr, mut_arr, ai_arr,
               map_table, commander_table, mutation_table, ai_table):
    mesh = plsc.VectorSubcoreMesh(core_axis_name="c", subcore_axis_name="s")
    f32 = jnp.float32

    @functools.partial(
        pl.kernel,
        out_type=(
            jax.ShapeDtypeStruct((B, MAP_DIM), f32),
            jax.ShapeDtypeStruct((B, CMD_DIM), f32),
            jax.ShapeDtypeStruct((B, CMD_DIM), f32),
            jax.ShapeDtypeStruct((B, MUT_DIM), f32),
            jax.ShapeDtypeStruct((B, AI_DIM), f32),
        ),
        mesh=mesh,
        scratch_types=[
            pltpu.VMEM((CPW, CH), jnp.int32),
            pltpu.VMEM((CPW, 2, CH), jnp.int32),
            pltpu.VMEM((CPW, MUT_SLOTS, CH), jnp.int32),
            pltpu.VMEM((CPW, CH), jnp.int32),
            pltpu.VMEM((RPW, MAP_DIM), f32),
            pltpu.VMEM((RPW, CMD_DIM), f32),
            pltpu.VMEM((RPW, CMD_DIM), f32),
            pltpu.VMEM((RPW, AI_DIM), f32),
            pltpu.VMEM((RPW, MUT_DIM), f32),
            pltpu.SemaphoreType.DMA,
            pltpu.SemaphoreType.DMA,
        ],
    )
    def k(map_i, cmd_i, mut_i, ai_i, mt, ct, mutt, at_,
          o_map, o_c0, o_c1, o_mut, o_ai,
          idx_map, idx_cmd, idx_mut, idx_ai,
          r_map, r_c0, r_c1, r_ai, acc, sem_g, sem_m):
        wid = lax.axis_index("s") * 2 + lax.axis_index("c")
        cbase = wid * CPW
        rbase = wid * RPW
        pltpu.sync_copy(map_i.at[pl.ds(cbase, CPW)], idx_map)
        pltpu.sync_copy(cmd_i.at[pl.ds(cbase, CPW)], idx_cmd)
        pltpu.sync_copy(mut_i.at[pl.ds(cbase, CPW)], idx_mut)
        pltpu.sync_copy(ai_i.at[pl.ds(cbase, CPW)], idx_ai)

        cps = []
        for j in range(CPW):
            d = pl.ds(j * CH, CH)
            cps.append(pltpu.async_copy(mt.at[idx_map.at[j]], r_map.at[d], sem_g))
            cps.append(pltpu.async_copy(ct.at[idx_cmd.at[j, 0]], r_c0.at[d], sem_g))
            cps.append(pltpu.async_copy(ct.at[idx_cmd.at[j, 1]], r_c1.at[d], sem_g))
            cps.append(pltpu.async_copy(at_.at[idx_ai.at[j]], r_ai.at[d], sem_g))

        m0 = [pltpu.async_copy(mutt.at[idx_mut.at[j, 0]],
                               acc.at[pl.ds(j * CH, CH)], sem_m)
              for j in range(CPW)]
        for cp in m0:
            cp.wait()

        def slot_body(s, carry):
            ms = [pltpu.async_copy(mutt.at[idx_mut.at[j, s]],
                                   acc.at[pl.ds(j * CH, CH)], sem_m, add=True)
                  for j in range(CPW)]
            for cp in ms:
                cp.wait()
            return carry

        lax.fori_loop(1, MUT_SLOTS, slot_body, 0)

        for cp in cps:
            cp.wait()

        pltpu.sync_copy(r_map, o_map.at[pl.ds(rbase, RPW)])
        pltpu.sync_copy(r_c0, o_c0.at[pl.ds(rbase, RPW)])
        pltpu.sync_copy(r_c1, o_c1.at[pl.ds(rbase, RPW)])
        pltpu.sync_copy(acc, o_mut.at[pl.ds(rbase, RPW)])
        pltpu.sync_copy(r_ai, o_ai.at[pl.ds(rbase, RPW)])

    return k(map_arr, cmd_arr, mut_arr, ai_arr,
             map_table, commander_table, mutation_table, ai_table)


def _tc_combine(map_e, c0, c1, mut_sum, ai_e, w0t, w1t, b2):
    BM = 2048
    grid = (B // BM,)

    def body(m_ref, c0_ref, c1_ref, mu_ref, a_ref, w0_ref, w1_ref, b_ref, o_ref):
        cmd = (
            jnp.dot(c0_ref[...], w0_ref[...], preferred_element_type=jnp.float32)
            + jnp.dot(c1_ref[...], w1_ref[...], preferred_element_type=jnp.float32)
            + b_ref[...]
        )
        o_ref[...] = jnp.concatenate(
            [m_ref[...], cmd, mu_ref[...] * (1.0 / MUT_SLOTS), a_ref[...]],
            axis=1,
        )

    return pl.pallas_call(
        body,
        grid=grid,
        in_specs=[
            pl.BlockSpec((BM, MAP_DIM), lambda i: (i, 0)),
            pl.BlockSpec((BM, CMD_DIM), lambda i: (i, 0)),
            pl.BlockSpec((BM, CMD_DIM), lambda i: (i, 0)),
            pl.BlockSpec((BM, MUT_DIM), lambda i: (i, 0)),
            pl.BlockSpec((BM, AI_DIM), lambda i: (i, 0)),
            pl.BlockSpec((CMD_DIM, CMD_DIM), lambda i: (0, 0)),
            pl.BlockSpec((CMD_DIM, CMD_DIM), lambda i: (0, 0)),
            pl.BlockSpec((1, CMD_DIM), lambda i: (0, 0)),
        ],
        out_specs=pl.BlockSpec((BM, MAP_DIM + CMD_DIM + MUT_DIM + AI_DIM),
                               lambda i: (i, 0)),
        out_shape=jax.ShapeDtypeStruct(
            (B, MAP_DIM + CMD_DIM + MUT_DIM + AI_DIM), jnp.float32),
    )(map_e, c0, c1, mut_sum, ai_e, w0t, w1t, b2)


def kernel(map_ids, commander_ids, mutation_ids, ai_ids,
           map_table, commander_table, mutation_table, ai_table,
           combine_W, combine_b):
    nch = B // CH
    map_arr = map_ids.astype(jnp.int32).reshape(nch, CH)
    cmd_arr = commander_ids.astype(jnp.int32).T.reshape(2, nch, CH).transpose(1, 0, 2)
    mut_arr = mutation_ids.astype(jnp.int32).T.reshape(MUT_SLOTS, nch, CH).transpose(1, 0, 2)
    ai_arr = ai_ids.astype(jnp.int32).reshape(nch, CH)

    map_e, c0, c1, mut_sum, ai_e = _sc_gather(
        map_arr, cmd_arr, mut_arr, ai_arr,
        map_table, commander_table, mutation_table, ai_table)

    w0t = combine_W[:, :CMD_DIM].T
    w1t = combine_W[:, CMD_DIM:].T
    b2 = combine_b.reshape(1, CMD_DIM)
    return _tc_combine(map_e, c0, c1, mut_sum, ai_e, w0t, w1t, b2)


# speedup vs baseline: 6.0584x; 6.0584x over previous
import functools

import jax
import jax.numpy as jnp
from jax import lax
from jax.experimental import pallas as pl
from jax.experimental.pallas import tpu as pltpu
from jax.experimental.pallas import tpu_sc as plsc

B = 16384
MUT_SLOTS = 20
CH = 128
NW = 32
CPW = (B // NW) // CH
RPW = B // NW

MAP_DIM = 32
CMD_DIM = 48
MUT_DIM = 48
AI_DIM = 16


def _sc_gather(map_arr, cmd_arr, mut_arr, ai_arr,
               map_table, commander_table, mutation_table, ai_table):
    mesh = plsc.VectorSubcoreMesh(core_axis_name="c", subcore_axis_name="s")
    f32 = jnp.float32

    @functools.partial(
        pl.kernel,
        out_type=(
            jax.ShapeDtypeStruct((B, MAP_DIM), f32),
            jax.ShapeDtypeStruct((B, CMD_DIM), f32),
            jax.ShapeDtypeStruct((B, CMD_DIM), f32),
            jax.ShapeDtypeStruct((B, MUT_DIM), f32),
            jax.ShapeDtypeStruct((B, AI_DIM), f32),
        ),
        mesh=mesh,
        compiler_params=pltpu.CompilerParams(use_tc_tiling_on_sc=False),
        scratch_types=[
            pltpu.VMEM((CPW, CH), jnp.int32),
            pltpu.VMEM((CPW, 2, CH), jnp.int32),
            pltpu.VMEM((CPW, MUT_SLOTS, CH), jnp.int32),
            pltpu.VMEM((CPW, CH), jnp.int32),
            pltpu.VMEM((RPW, MAP_DIM), f32),
            pltpu.VMEM((RPW, CMD_DIM), f32),
            pltpu.VMEM((RPW, CMD_DIM), f32),
            pltpu.VMEM((RPW, AI_DIM), f32),
            pltpu.VMEM((RPW, MUT_DIM), f32),
            pltpu.SemaphoreType.DMA,
            pltpu.SemaphoreType.DMA,
        ],
    )
    def k(map_i, cmd_i, mut_i, ai_i, mt, ct, mutt, at_,
          o_map, o_c0, o_c1, o_mut, o_ai,
          idx_map, idx_cmd, idx_mut, idx_ai,
          r_map, r_c0, r_c1, r_ai, acc, sem_g, sem_m):
        wid = lax.axis_index("s") * 2 + lax.axis_index("c")
        cbase = wid * CPW
        rbase = wid * RPW
        pltpu.sync_copy(map_i.at[pl.ds(cbase, CPW)], idx_map)
        pltpu.sync_copy(cmd_i.at[pl.ds(cbase, CPW)], idx_cmd)
        pltpu.sync_copy(mut_i.at[pl.ds(cbase, CPW)], idx_mut)
        pltpu.sync_copy(ai_i.at[pl.ds(cbase, CPW)], idx_ai)

        cps = []
        for j in range(CPW):
            d = pl.ds(j * CH, CH)
            cps.append(pltpu.async_copy(mt.at[idx_map.at[j]], r_map.at[d], sem_g))
            cps.append(pltpu.async_copy(ct.at[idx_cmd.at[j, 0]], r_c0.at[d], sem_g))
            cps.append(pltpu.async_copy(ct.at[idx_cmd.at[j, 1]], r_c1.at[d], sem_g))
            cps.append(pltpu.async_copy(at_.at[idx_ai.at[j]], r_ai.at[d], sem_g))

        m0 = [pltpu.async_copy(mutt.at[idx_mut.at[j, 0]],
                               acc.at[pl.ds(j * CH, CH)], sem_m)
              for j in range(CPW)]
        for cp in m0:
            cp.wait()

        def slot_body(s, carry):
            ms = [pltpu.async_copy(mutt.at[idx_mut.at[j, s]],
                                   acc.at[pl.ds(j * CH, CH)], sem_m, add=True)
                  for j in range(CPW)]
            for cp in ms:
                cp.wait()
            return carry

        lax.fori_loop(1, MUT_SLOTS, slot_body, 0)

        for cp in cps:
            cp.wait()

        pltpu.sync_copy(r_map, o_map.at[pl.ds(rbase, RPW)])
        pltpu.sync_copy(r_c0, o_c0.at[pl.ds(rbase, RPW)])
        pltpu.sync_copy(r_c1, o_c1.at[pl.ds(rbase, RPW)])
        pltpu.sync_copy(acc, o_mut.at[pl.ds(rbase, RPW)])
        pltpu.sync_copy(r_ai, o_ai.at[pl.ds(rbase, RPW)])

    return k(map_arr, cmd_arr, mut_arr, ai_arr,
             map_table, commander_table, mutation_table, ai_table)


def _tc_combine(map_e, c0, c1, mut_sum, ai_e, w0t, w1t, b2):
    BM = 2048
    grid = (B // BM,)

    def body(m_ref, c0_ref, c1_ref, mu_ref, a_ref, w0_ref, w1_ref, b_ref, o_ref):
        cmd = (
            jnp.dot(c0_ref[...], w0_ref[...], preferred_element_type=jnp.float32)
            + jnp.dot(c1_ref[...], w1_ref[...], preferred_element_type=jnp.float32)
            + b_ref[...]
        )
        o_ref[...] = jnp.concatenate(
            [m_ref[...], cmd, mu_ref[...] * (1.0 / MUT_SLOTS), a_ref[...]],
            axis=1,
        )

    return pl.pallas_call(
        body,
        grid=grid,
        in_specs=[
            pl.BlockSpec((BM, MAP_DIM), lambda i: (i, 0)),
            pl.BlockSpec((BM, CMD_DIM), lambda i: (i, 0)),
            pl.BlockSpec((BM, CMD_DIM), lambda i: (i, 0)),
            pl.BlockSpec((BM, MUT_DIM), lambda i: (i, 0)),
            pl.BlockSpec((BM, AI_DIM), lambda i: (i, 0)),
            pl.BlockSpec((CMD_DIM, CMD_DIM), lambda i: (0, 0)),
            pl.BlockSpec((CMD_DIM, CMD_DIM), lambda i: (0, 0)),
            pl.BlockSpec((1, CMD_DIM), lambda i: (0, 0)),
        ],
        out_specs=pl.BlockSpec((BM, MAP_DIM + CMD_DIM + MUT_DIM + AI_DIM),
                               lambda i: (i, 0)),
        out_shape=jax.ShapeDtypeStruct(
            (B, MAP_DIM + CMD_DIM + MUT_DIM + AI_DIM), jnp.float32),
    )(map_e, c0, c1, mut_sum, ai_e, w0t, w1t, b2)


def kernel(map_ids, commander_ids, mutation_ids, ai_ids,
           map_table, commander_table, mutation_table, ai_table,
           combine_W, combine_b):
    nch = B // CH
    map_arr = map_ids.astype(jnp.int32).reshape(nch, CH)
    cmd_arr = commander_ids.astype(jnp.int32).T.reshape(2, nch, CH).transpose(1, 0, 2)
    mut_arr = mutation_ids.astype(jnp.int32).T.reshape(MUT_SLOTS, nch, CH).transpose(1, 0, 2)
    ai_arr = ai_ids.astype(jnp.int32).reshape(nch, CH)

    map_e, c0, c1, mut_sum, ai_e = _sc_gather(
        map_arr, cmd_arr, mut_arr, ai_arr,
        map_table, commander_table, mutation_table, ai_table)

    w0t = combine_W[:, :CMD_DIM].T
    w1t = combine_W[:, CMD_DIM:].T
    b2 = combine_b.reshape(1, CMD_DIM)
    return _tc_combine(map_e, c0, c1, mut_sum, ai_e, w0t, w1t, b2)
